# baseline (speedup 1.0000x reference)
import functools

import jax
import jax.numpy as jnp
from jax import lax
from jax.experimental import pallas as pl
from jax.experimental.pallas import tpu as pltpu
from jax.experimental.pallas import tpu_sc as plsc

EMBED = 64
ROWPAD = 65
N_DAY, N_HOUR, N_HOL = 7, 24, 2
N_COMB = N_DAY * N_HOUR * N_HOL

_NC = 2
_NS = 16
_NW = _NC * _NS
_LANES = 16


def _ctable_body(day_ref, hour_ref, hol_ref, out_ref):
    hol = hol_ref[:]
    for d in range(N_DAY):
        for h in range(N_HOUR):
            out_ref[pl.ds(d * (N_HOUR * N_HOL) + h * N_HOL, N_HOL),
                    pl.ds(0, EMBED)] = (
                day_ref[pl.ds(d, 1), :] + hour_ref[pl.ds(h, 1), :] + hol
            )
    out_ref[:, pl.ds(EMBED, ROWPAD - EMBED)] = jnp.zeros(
        (N_COMB, ROWPAD - EMBED), jnp.float32)


def _build_ctable(day_table, hour_table, holiday_table):
    return pl.pallas_call(
        _ctable_body,
        out_shape=jax.ShapeDtypeStruct((N_COMB, ROWPAD), jnp.float32),
    )(day_table, hour_table, holiday_table)


def _make_sc_lookup(n_batch, n_seq):
    bw = n_batch // _NW
    n_grp = bw // _LANES
    mesh = plsc.VectorSubcoreMesh(core_axis_name="c", subcore_axis_name="s")

    @functools.partial(
        pl.kernel,
        mesh=mesh,
        compiler_params=pltpu.CompilerParams(use_tc_tiling_on_sc=False,
                                             needs_layout_passes=False),
        out_type=jax.ShapeDtypeStruct((n_seq, EMBED, n_batch), jnp.float32),
        scratch_types=[
            pltpu.VMEM((N_COMB * ROWPAD,), jnp.float32),
            pltpu.VMEM((2, bw), jnp.int32),
            pltpu.VMEM((2, bw), jnp.int32),
            pltpu.VMEM((2, bw), jnp.int32),
            pltpu.VMEM((bw,), jnp.int32),
            pltpu.VMEM((2, EMBED, bw), jnp.float32),
            pltpu.SemaphoreType.DMA,
            pltpu.SemaphoreType.DMA,
            pltpu.SemaphoreType.DMA,
            pltpu.SemaphoreType.DMA,
            pltpu.SemaphoreType.DMA,
        ],
    )
    def sc_kernel(ctable_hbm, day_hbm, hour_hbm, hol_hbm, out_hbm,
                  ctab_v, day_v, hour_v, hol_v, addr_v, stage_v,
                  tsem, isem0, isem1, osem0, osem1):
        isems = (isem0, isem1)
        osems = (osem0, osem1)
        wid = lax.axis_index("s") * _NC + lax.axis_index("c")
        b0 = wid * bw

        tcopy = pltpu.async_copy(ctable_hbm, ctab_v, tsem)

        def issue_idx(l, b):
            lc = jnp.minimum(l, n_seq - 1)
            pltpu.async_copy(day_hbm.at[lc, pl.ds(b0, bw)], day_v.at[b],
                             isems[b])
            pltpu.async_copy(hour_hbm.at[lc, pl.ds(b0, bw)], hour_v.at[b],
                             isems[b])
            pltpu.async_copy(hol_hbm.at[lc, pl.ds(b0, bw)], hol_v.at[b],
                             isems[b])

        def wait_idx(b):
            for ref in (day_v, hour_v, hol_v):
                pltpu.make_async_copy(day_hbm.at[0, pl.ds(b0, bw)],
                                      ref.at[b], isems[b]).wait()

        def issue_out(l, b):
            pltpu.async_copy(stage_v.at[b],
                             out_hbm.at[l, :, pl.ds(b0, bw)], osems[b])

        def wait_out(b):
            pltpu.make_async_copy(stage_v.at[b],
                                  out_hbm.at[0, :, pl.ds(b0, bw)],
                                  osems[b]).wait()

        issue_idx(0, 0)
        issue_idx(1, 1)
        issue_out(0, 0)
        issue_out(1, 1)

        tcopy.wait()

        def step(l, b):
            wait_idx(b)
            for g in range(n_grp):
                s = pl.ds(g * _LANES, _LANES)
                fused = (day_v[b, s] * (N_HOUR * N_HOL)
                         + hour_v[b, s] * N_HOL + hol_v[b, s])
                addr_v[s] = fused * ROWPAD
            issue_idx(l + 2, b)
            wait_out(b)

            def grp_loop(g, carry):
                gb = g * _LANES
                rowaddr = addr_v[pl.ds(gb, _LANES)]
                for c in range(EMBED):
                    vals = plsc.load_gather(ctab_v, [rowaddr + c])
                    stage_v[b, c, pl.ds(gb, _LANES)] = vals
                return carry

            lax.fori_loop(0, n_grp, grp_loop, 0)
            issue_out(l, b)

        def body(k, carry):
            step(2 * k, 0)
            step(2 * k + 1, 1)
            return carry

        lax.fori_loop(0, n_seq // 2, body, 0)
        wait_idx(0)
        wait_idx(1)
        wait_out(0)
        wait_out(1)

    return sc_kernel


def kernel(day_of_week, hour_of_day, is_holiday, day_table, hour_table,
           holiday_table):
    b, l = day_of_week.shape
    dow = day_of_week.T.astype(jnp.int32)
    hod = hour_of_day.T.astype(jnp.int32)
    hol = is_holiday.T.astype(jnp.int32)
    ctable = _build_ctable(day_table, hour_table, holiday_table).reshape(-1)
    out_t = _make_sc_lookup(b, l)(ctable, dow, hod, hol)
    return jnp.transpose(out_t, (2, 0, 1))


# speedup vs baseline: 1.8864x; 1.8864x over previous
import functools

import jax
import jax.numpy as jnp
from jax import lax
from jax.experimental import pallas as pl
from jax.experimental.pallas import tpu as pltpu
from jax.experimental.pallas import tpu_sc as plsc

EMBED = 64
ROWPAD = 65
N_DAY, N_HOUR, N_HOL = 7, 24, 2
N_COMB = N_DAY * N_HOUR * N_HOL

_NC = 2
_NS = 16
_NW = _NC * _NS
_LANES = 16


def _ctable_body(day_ref, hour_ref, hol_ref, out_ref):
    hol = hol_ref[:]
    for d in range(N_DAY):
        for h in range(N_HOUR):
            out_ref[pl.ds(d * (N_HOUR * N_HOL) + h * N_HOL, N_HOL),
                    pl.ds(0, EMBED)] = (
                day_ref[pl.ds(d, 1), :] + hour_ref[pl.ds(h, 1), :] + hol
            )
    out_ref[:, pl.ds(EMBED, ROWPAD - EMBED)] = jnp.zeros(
        (N_COMB, ROWPAD - EMBED), jnp.float32)


def _build_ctable(day_table, hour_table, holiday_table):
    return pl.pallas_call(
        _ctable_body,
        out_shape=jax.ShapeDtypeStruct((N_COMB, ROWPAD), jnp.float32),
    )(day_table, hour_table, holiday_table)


def _make_sc_lookup(n_batch, n_seq):
    bw = n_batch // _NW
    n_grp = bw // _LANES
    mesh = plsc.VectorSubcoreMesh(core_axis_name="c", subcore_axis_name="s")

    @functools.partial(
        pl.kernel,
        mesh=mesh,
        compiler_params=pltpu.CompilerParams(use_tc_tiling_on_sc=False,
                                             needs_layout_passes=False),
        out_type=jax.ShapeDtypeStruct((n_seq, EMBED, n_batch), jnp.float32),
        scratch_types=[
            pltpu.VMEM((N_COMB * ROWPAD,), jnp.float32),
            pltpu.VMEM((2, bw), jnp.int32),
            pltpu.VMEM((2, bw), jnp.int32),
            pltpu.VMEM((2, bw), jnp.int32),
            pltpu.VMEM((bw,), jnp.int32),
            pltpu.VMEM((2, EMBED, bw), jnp.float32),
            pltpu.SemaphoreType.DMA,
            pltpu.SemaphoreType.DMA,
            pltpu.SemaphoreType.DMA,
            pltpu.SemaphoreType.DMA,
            pltpu.SemaphoreType.DMA,
        ],
    )
    def sc_kernel(ctable_hbm, day_hbm, hour_hbm, hol_hbm, out_hbm,
                  ctab_v, day_v, hour_v, hol_v, addr_v, stage_v,
                  tsem, isem0, isem1, osem0, osem1):
        isems = (isem0, isem1)
        osems = (osem0, osem1)
        wid = lax.axis_index("s") * _NC + lax.axis_index("c")
        b0 = wid * bw

        tcopy = pltpu.async_copy(ctable_hbm, ctab_v, tsem)

        def issue_idx(l, b):
            lc = jnp.minimum(l, n_seq - 1)
            pltpu.async_copy(day_hbm.at[lc, pl.ds(b0, bw)], day_v.at[b],
                             isems[b])
            pltpu.async_copy(hour_hbm.at[lc, pl.ds(b0, bw)], hour_v.at[b],
                             isems[b])
            pltpu.async_copy(hol_hbm.at[lc, pl.ds(b0, bw)], hol_v.at[b],
                             isems[b])

        def wait_idx(b):
            for ref in (day_v, hour_v, hol_v):
                pltpu.make_async_copy(day_hbm.at[0, pl.ds(b0, bw)],
                                      ref.at[b], isems[b]).wait()

        def issue_out(l, b):
            pltpu.async_copy(stage_v.at[b],
                             out_hbm.at[l, :, pl.ds(b0, bw)], osems[b])

        def wait_out(b):
            pltpu.make_async_copy(stage_v.at[b],
                                  out_hbm.at[0, :, pl.ds(b0, bw)],
                                  osems[b]).wait()

        issue_idx(0, 0)
        issue_idx(1, 1)
        issue_out(0, 0)
        issue_out(1, 1)

        tcopy.wait()

        def step(l, b):
            wait_idx(b)
            for g in range(n_grp):
                s = pl.ds(g * _LANES, _LANES)
                fused = (day_v[b, s] * (N_HOUR * N_HOL)
                         + hour_v[b, s] * N_HOL + hol_v[b, s])
                addr_v[s] = fused * ROWPAD
            issue_idx(l + 2, b)
            wait_out(b)

            def grp_loop(g, carry):
                gb = g * _LANES
                rowaddr = addr_v[pl.ds(gb, _LANES)]
                for c0 in range(0, EMBED, 8):
                    vals = [plsc.load_gather(ctab_v, [rowaddr + (c0 + i)])
                            for i in range(8)]
                    for i in range(8):
                        stage_v[b, c0 + i, pl.ds(gb, _LANES)] = vals[i]
                return carry

            lax.fori_loop(0, n_grp, grp_loop, 0)
            issue_out(l, b)

        def body(k, carry):
            step(2 * k, 0)
            step(2 * k + 1, 1)
            return carry

        lax.fori_loop(0, n_seq // 2, body, 0)
        wait_idx(0)
        wait_idx(1)
        wait_out(0)
        wait_out(1)

    return sc_kernel


def kernel(day_of_week, hour_of_day, is_holiday, day_table, hour_table,
           holiday_table):
    b, l = day_of_week.shape
    dow = day_of_week.T.astype(jnp.int32)
    hod = hour_of_day.T.astype(jnp.int32)
    hol = is_holiday.T.astype(jnp.int32)
    ctable = _build_ctable(day_table, hour_table, holiday_table).reshape(-1)
    out_t = _make_sc_lookup(b, l)(ctable, dow, hod, hol)
    return jnp.transpose(out_t, (2, 0, 1))


# speedup vs baseline: 4.7650x; 2.5260x over previous
import functools

import jax
import jax.numpy as jnp
from jax import lax
from jax.experimental import pallas as pl
from jax.experimental.pallas import tpu as pltpu
from jax.experimental.pallas import tpu_sc as plsc

EMBED = 64
ROWPAD = 65
N_DAY, N_HOUR, N_HOL = 7, 24, 2
N_COMB = N_DAY * N_HOUR * N_HOL

_NC = 2
_NS = 16
_NW = _NC * _NS
_LANES = 16


def _ctable_body(day_ref, hour_ref, hol_ref, out_ref):
    hol = hol_ref[:]
    for d in range(N_DAY):
        for h in range(N_HOUR):
            out_ref[pl.ds(d * (N_HOUR * N_HOL) + h * N_HOL, N_HOL),
                    pl.ds(0, EMBED)] = (
                day_ref[pl.ds(d, 1), :] + hour_ref[pl.ds(h, 1), :] + hol
            )
    out_ref[:, pl.ds(EMBED, ROWPAD - EMBED)] = jnp.zeros(
        (N_COMB, ROWPAD - EMBED), jnp.float32)


def _build_ctable(day_table, hour_table, holiday_table):
    return pl.pallas_call(
        _ctable_body,
        out_shape=jax.ShapeDtypeStruct((N_COMB, ROWPAD), jnp.float32),
    )(day_table, hour_table, holiday_table)


def _make_sc_lookup(n_batch, n_seq):
    bw = n_batch // _NW
    n_grp = bw // _LANES
    ct = EMBED // 8
    bt = n_batch // 128
    btw = bw // 128
    mesh = plsc.VectorSubcoreMesh(core_axis_name="c", subcore_axis_name="s")

    @functools.partial(
        pl.kernel,
        mesh=mesh,
        compiler_params=pltpu.CompilerParams(use_tc_tiling_on_sc=False,
                                             needs_layout_passes=False),
        out_type=jax.ShapeDtypeStruct((n_seq, ct, bt, 8, 128), jnp.float32),
        scratch_types=[
            pltpu.VMEM((N_COMB * ROWPAD,), jnp.float32),
            pltpu.VMEM((2, bw), jnp.int32),
            pltpu.VMEM((2, bw), jnp.int32),
            pltpu.VMEM((2, bw), jnp.int32),
            pltpu.VMEM((bw,), jnp.int32),
            pltpu.VMEM((2, ct, btw, 8, 128), jnp.float32),
            pltpu.SemaphoreType.DMA,
            pltpu.SemaphoreType.DMA,
            pltpu.SemaphoreType.DMA,
            pltpu.SemaphoreType.DMA,
            pltpu.SemaphoreType.DMA,
        ],
    )
    def sc_kernel(ctable_hbm, day_hbm, hour_hbm, hol_hbm, out_hbm,
                  ctab_v, day_v, hour_v, hol_v, addr_v, stage_v,
                  tsem, isem0, isem1, osem0, osem1):
        isems = (isem0, isem1)
        osems = (osem0, osem1)
        wid = lax.axis_index("s") * _NC + lax.axis_index("c")
        b0 = wid * bw

        tcopy = pltpu.async_copy(ctable_hbm, ctab_v, tsem)

        def issue_idx(l, b):
            lc = jnp.minimum(l, n_seq - 1)
            pltpu.async_copy(day_hbm.at[lc, pl.ds(b0, bw)], day_v.at[b],
                             isems[b])
            pltpu.async_copy(hour_hbm.at[lc, pl.ds(b0, bw)], hour_v.at[b],
                             isems[b])
            pltpu.async_copy(hol_hbm.at[lc, pl.ds(b0, bw)], hol_v.at[b],
                             isems[b])

        def wait_idx(b):
            for ref in (day_v, hour_v, hol_v):
                pltpu.make_async_copy(day_hbm.at[0, pl.ds(b0, bw)],
                                      ref.at[b], isems[b]).wait()

        def issue_out(l, b):
            pltpu.async_copy(stage_v.at[b],
                             out_hbm.at[l, :, pl.ds(wid * btw, btw)],
                             osems[b])

        def wait_out(b):
            pltpu.make_async_copy(stage_v.at[b],
                                  out_hbm.at[0, :, pl.ds(wid * btw, btw)],
                                  osems[b]).wait()

        issue_idx(0, 0)
        issue_idx(1, 1)
        issue_out(0, 0)
        issue_out(1, 1)

        tcopy.wait()

        def step(l, b):
            wait_idx(b)
            for g in range(n_grp):
                s = pl.ds(g * _LANES, _LANES)
                fused = (day_v[b, s] * (N_HOUR * N_HOL)
                         + hour_v[b, s] * N_HOL + hol_v[b, s])
                addr_v[s] = fused * ROWPAD
            issue_idx(l + 2, b)
            wait_out(b)

            def grp_loop(g, carry):
                gb = g * _LANES
                btl = g // 8
                bw0 = (g % 8) * _LANES
                rowaddr = addr_v[pl.ds(gb, _LANES)]
                for c0 in range(0, EMBED, 8):
                    vals = [plsc.load_gather(ctab_v, [rowaddr + (c0 + i)])
                            for i in range(8)]
                    for i in range(8):
                        c = c0 + i
                        stage_v[b, c // 8, btl, c % 8, pl.ds(bw0, _LANES)] = (
                            vals[i])
                return carry

            lax.fori_loop(0, n_grp, grp_loop, 0)
            issue_out(l, b)

        def body(k, carry):
            step(2 * k, 0)
            step(2 * k + 1, 1)
            return carry

        lax.fori_loop(0, n_seq // 2, body, 0)
        wait_idx(0)
        wait_idx(1)
        wait_out(0)
        wait_out(1)

    return sc_kernel


def kernel(day_of_week, hour_of_day, is_holiday, day_table, hour_table,
           holiday_table):
    b, l = day_of_week.shape
    dow = day_of_week.T.astype(jnp.int32)
    hod = hour_of_day.T.astype(jnp.int32)
    hol = is_holiday.T.astype(jnp.int32)
    ctable = _build_ctable(day_table, hour_table, holiday_table).reshape(-1)
    out5 = _make_sc_lookup(b, l)(ctable, dow, hod, hol)
    out = jnp.transpose(out5, (0, 1, 3, 2, 4)).reshape(l, EMBED, b)
    return jnp.transpose(out, (2, 0, 1))


# speedup vs baseline: 5.1812x; 1.0874x over previous
import functools

import jax
import jax.numpy as jnp
from jax import lax
from jax.experimental import pallas as pl
from jax.experimental.pallas import tpu as pltpu
from jax.experimental.pallas import tpu_sc as plsc

EMBED = 64
ROWPAD = 65
N_DAY, N_HOUR, N_HOL = 7, 24, 2
N_COMB = N_DAY * N_HOUR * N_HOL

_NC = 2
_NS = 16
_NW = _NC * _NS
_LANES = 16


def _ctable_body(day_ref, hour_ref, hol_ref, out_ref):
    hol = hol_ref[:]
    for d in range(N_DAY):
        for h in range(N_HOUR):
            out_ref[pl.ds(d * (N_HOUR * N_HOL) + h * N_HOL, N_HOL),
                    pl.ds(0, EMBED)] = (
                day_ref[pl.ds(d, 1), :] + hour_ref[pl.ds(h, 1), :] + hol
            )
    out_ref[:, pl.ds(EMBED, ROWPAD - EMBED)] = jnp.zeros(
        (N_COMB, ROWPAD - EMBED), jnp.float32)


def _build_ctable(day_table, hour_table, holiday_table):
    return pl.pallas_call(
        _ctable_body,
        out_shape=jax.ShapeDtypeStruct((N_COMB, ROWPAD), jnp.float32),
    )(day_table, hour_table, holiday_table)


def _make_sc_lookup(n_batch, n_seq):
    bw = n_batch // _NW
    n_grp = bw // _LANES
    ct = EMBED // 8
    bt = n_batch // 128
    btw = bw // 128
    mesh = plsc.VectorSubcoreMesh(core_axis_name="c", subcore_axis_name="s")

    @functools.partial(
        pl.kernel,
        mesh=mesh,
        compiler_params=pltpu.CompilerParams(use_tc_tiling_on_sc=False,
                                             needs_layout_passes=False),
        out_type=jax.ShapeDtypeStruct((n_seq, ct, bt, 8, 128), jnp.float32),
        scratch_types=[
            pltpu.VMEM((N_COMB * ROWPAD,), jnp.float32),
            pltpu.VMEM((2, btw, 128), jnp.int32),
            pltpu.VMEM((2, btw, 128), jnp.int32),
            pltpu.VMEM((2, btw, 128), jnp.int32),
            pltpu.VMEM((bw,), jnp.int32),
            pltpu.VMEM((2, ct, btw, 8, 128), jnp.float32),
            pltpu.SemaphoreType.DMA,
            pltpu.SemaphoreType.DMA,
            pltpu.SemaphoreType.DMA,
            pltpu.SemaphoreType.DMA,
            pltpu.SemaphoreType.DMA,
        ],
    )
    def sc_kernel(ctable_hbm, day_hbm, hour_hbm, hol_hbm, out_hbm,
                  ctab_v, day_v, hour_v, hol_v, addr_v, stage_v,
                  tsem, isem0, isem1, osem0, osem1):
        isems = (isem0, isem1)
        osems = (osem0, osem1)
        wid = lax.axis_index("s") * _NC + lax.axis_index("c")
        b0 = wid * bw

        tcopy = pltpu.async_copy(ctable_hbm, ctab_v, tsem)

        def issue_idx(l, b):
            lc = jnp.minimum(l, n_seq - 1)
            lt = lc // 8
            lw = lc % 8
            bsel = pl.ds(wid * btw, btw)
            pltpu.async_copy(day_hbm.at[lt, bsel, lw], day_v.at[b], isems[b])
            pltpu.async_copy(hour_hbm.at[lt, bsel, lw], hour_v.at[b], isems[b])
            pltpu.async_copy(hol_hbm.at[lt, bsel, lw], hol_v.at[b], isems[b])

        def wait_idx(b):
            for ref in (day_v, hour_v, hol_v):
                pltpu.make_async_copy(
                    day_hbm.at[0, pl.ds(wid * btw, btw), 0],
                    ref.at[b], isems[b]).wait()

        def issue_out(l, b):
            pltpu.async_copy(stage_v.at[b],
                             out_hbm.at[l, :, pl.ds(wid * btw, btw)],
                             osems[b])

        def wait_out(b):
            pltpu.make_async_copy(stage_v.at[b],
                                  out_hbm.at[0, :, pl.ds(wid * btw, btw)],
                                  osems[b]).wait()

        issue_idx(0, 0)
        issue_idx(1, 1)
        issue_out(0, 0)
        issue_out(1, 1)

        tcopy.wait()

        def step(l, b):
            wait_idx(b)
            for g in range(n_grp):
                t, o = g // 8, (g % 8) * _LANES
                s = pl.ds(o, _LANES)
                fused = (day_v[b, t, s] * (N_HOUR * N_HOL)
                         + hour_v[b, t, s] * N_HOL + hol_v[b, t, s])
                addr_v[pl.ds(g * _LANES, _LANES)] = fused * ROWPAD
            issue_idx(l + 2, b)
            wait_out(b)

            def grp_loop(g, carry):
                gb = g * _LANES
                btl = g // 8
                bw0 = (g % 8) * _LANES
                rowaddr = addr_v[pl.ds(gb, _LANES)]
                for c0 in range(0, EMBED, 8):
                    vals = [plsc.load_gather(ctab_v, [rowaddr + (c0 + i)])
                            for i in range(8)]
                    for i in range(8):
                        c = c0 + i
                        stage_v[b, c // 8, btl, c % 8, pl.ds(bw0, _LANES)] = (
                            vals[i])
                return carry

            lax.fori_loop(0, n_grp, grp_loop, 0)
            issue_out(l, b)

        def body(k, carry):
            step(2 * k, 0)
            step(2 * k + 1, 1)
            return carry

        lax.fori_loop(0, n_seq // 2, body, 0)
        wait_idx(0)
        wait_idx(1)
        wait_out(0)
        wait_out(1)

    return sc_kernel


def kernel(day_of_week, hour_of_day, is_holiday, day_table, hour_table,
           holiday_table):
    b, l = day_of_week.shape

    def tiled(a):
        return (a.astype(jnp.int32).T
                .reshape(l // 8, 8, b // 128, 128)
                .transpose(0, 2, 1, 3))

    dow = tiled(day_of_week)
    hod = tiled(hour_of_day)
    hol = tiled(is_holiday)
    ctable = _build_ctable(day_table, hour_table, holiday_table).reshape(-1)
    out5 = _make_sc_lookup(b, l)(ctable, dow, hod, hol)
    out = jnp.transpose(out5, (0, 1, 3, 2, 4)).reshape(l, EMBED, b)
    return jnp.transpose(out, (2, 0, 1))
